# baseline (speedup 1.0000x reference)
import functools

import jax
import jax.numpy as jnp
from jax import lax
from jax.experimental import pallas as pl
from jax.experimental.pallas import tpu as pltpu
from jax.experimental.pallas import tpu_sc as plsc

D = 64
XR = 4096
COLS = 200
B = XR * COLS
NC = 2
NS = 16
NW = NC * NS
XRW = XR // NW
CPAD = 208
S0 = 128
S1 = COLS - S0
NCH = XRW // 2
SCALE = 8.0


def _sc_embed(x, table):
    mesh = plsc.VectorSubcoreMesh(
        core_axis_name="c", subcore_axis_name="s", num_cores=NC)

    @functools.partial(
        pl.kernel,
        mesh=mesh,
        compiler_params=pltpu.CompilerParams(use_tc_tiling_on_sc=False),
        out_type=jax.ShapeDtypeStruct((XR, COLS, D), jnp.float32),
        scratch_types=[
            pltpu.VMEM((2, 2, CPAD), jnp.int32),
            pltpu.VMEM((2, 2, COLS, D), jnp.float32),
            pltpu.SemaphoreType.DMA,
            pltpu.SemaphoreType.DMA,
            pltpu.SemaphoreType.DMA,
            pltpu.SemaphoreType.DMA,
        ],
    )
    def k(x_hbm, tbl_hbm, out_hbm, idx_v, rows, g0, g1, o0, o1):
        wid = lax.axis_index("s") * NC + lax.axis_index("c")
        r0 = wid * XRW
        gsem = (g0, g1)
        osem = (o0, o1)

        def fire_gathers(b, c):
            for u in range(2):
                xr = r0 + 2 * c + u
                pltpu.sync_copy(x_hbm.at[xr, :], idx_v.at[b, u, pl.ds(0, COLS)])
            for u in range(2):
                pltpu.async_copy(
                    tbl_hbm.at[idx_v.at[b, u, pl.ds(0, S0)]],
                    rows.at[b, u, pl.ds(0, S0), :], gsem[b])
                pltpu.async_copy(
                    tbl_hbm.at[idx_v.at[b, u, pl.ds(S0, S1)]],
                    rows.at[b, u, pl.ds(S0, S1), :], gsem[b])

        def wait_sem(sem):
            pltpu.make_async_copy(
                out_hbm.at[pl.ds(0, 2)], rows.at[0], sem).wait()

        def scale_chunk(b):
            for u in range(2):
                def grp(kk, cc, u=u):
                    iv = idx_v[b, u, pl.ds(kk * 16, 16)]
                    fv = jnp.where(iv == 0, jnp.float32(0.0),
                                   jnp.float32(SCALE))
                    for t in range(16):
                        f = fv[t]
                        rr = kk * 16 + t
                        for q in range(D // 16):
                            sl = pl.ds(q * 16, 16)
                            rows[b, u, rr, sl] = rows[b, u, rr, sl] * f
                    return cc

                lax.fori_loop(0, COLS // 16, grp, 0)
                iv = idx_v[b, u, pl.ds(192, 16)]
                fv = jnp.where(iv == 0, jnp.float32(0.0), jnp.float32(SCALE))
                for t in range(COLS - 16 * (COLS // 16)):
                    f = fv[t]
                    rr = 192 + t
                    for q in range(D // 16):
                        sl = pl.ds(q * 16, 16)
                        rows[b, u, rr, sl] = rows[b, u, rr, sl] * f

        def store_chunk(b, c):
            pltpu.async_copy(
                rows.at[b], out_hbm.at[pl.ds(r0 + 2 * c, 2)], osem[b])

        fire_gathers(0, 0)

        def body(p, carry):
            @pl.when(p > 0)
            def _():
                wait_sem(osem[1])

            fire_gathers(1, 2 * p + 1)

            wait_sem(gsem[0])
            scale_chunk(0)
            store_chunk(0, 2 * p)

            wait_sem(gsem[1])
            scale_chunk(1)
            store_chunk(1, 2 * p + 1)

            @pl.when(p < NCH // 2 - 1)
            def _():
                wait_sem(osem[0])
                fire_gathers(0, 2 * p + 2)

            return carry

        lax.fori_loop(0, NCH // 2, body, 0)
        wait_sem(osem[0])
        wait_sem(osem[1])

    return k(x, table)


def kernel(x, table):
    return _sc_embed(x, table)


# speedup vs baseline: 1.0308x; 1.0308x over previous
import functools

import jax
import jax.numpy as jnp
from jax import lax
from jax.experimental import pallas as pl
from jax.experimental.pallas import tpu as pltpu
from jax.experimental.pallas import tpu_sc as plsc

D = 64
XR = 4096
COLS = 200
B = XR * COLS
NC = 2
NS = 16
NW = NC * NS
BPW = B // NW
SUB = 128
CHUNK = 512
NSUB = CHUNK // SUB
NG = BPW // CHUNK
SCALE = 8.0


def _sc_embed(xT, table):
    mesh = plsc.VectorSubcoreMesh(
        core_axis_name="c", subcore_axis_name="s", num_cores=NC)

    @functools.partial(
        pl.kernel,
        mesh=mesh,
        compiler_params=pltpu.CompilerParams(use_tc_tiling_on_sc=False),
        out_type=jax.ShapeDtypeStruct((COLS, XR, D), jnp.float32),
        scratch_types=[
            pltpu.VMEM((2, CHUNK), jnp.int32),
            pltpu.VMEM((2, CHUNK, D), jnp.float32),
            pltpu.SemaphoreType.DMA,
            pltpu.SemaphoreType.DMA,
            pltpu.SemaphoreType.DMA,
            pltpu.SemaphoreType.DMA,
        ],
    )
    def k(xT_hbm, tbl_hbm, out_hbm, idx_v, rows, g0, g1, o0, o1):
        wid = lax.axis_index("s") * NC + lax.axis_index("c")
        base = wid * BPW
        gsem = (g0, g1)
        osem = (o0, o1)

        def fire_gathers(b, g):
            p0 = base + g * CHUNK
            jj = lax.shift_right_logical(p0, 12)
            ii = pl.multiple_of(lax.bitwise_and(p0, XR - 1), CHUNK)
            pltpu.sync_copy(
                xT_hbm.at[jj, pl.ds(ii, CHUNK)], idx_v.at[b])
            for j in range(NSUB):
                pltpu.async_copy(
                    tbl_hbm.at[idx_v.at[b, pl.ds(j * SUB, SUB)]],
                    rows.at[b, pl.ds(j * SUB, SUB), :], gsem[b])

        def wait_sem(sem):
            pltpu.make_async_copy(
                out_hbm.at[0, pl.ds(0, CHUNK)], rows.at[0], sem).wait()

        def scale_chunk(b):
            def grp(kk, cc):
                iv = idx_v[b, pl.ds(kk * 16, 16)]
                fv = jnp.where(iv == 0, jnp.float32(0.0), jnp.float32(SCALE))
                for t in range(16):
                    f = fv[t]
                    rr = kk * 16 + t
                    for q in range(D // 16):
                        sl = pl.ds(q * 16, 16)
                        rows[b, rr, sl] = rows[b, rr, sl] * f
                return cc

            lax.fori_loop(0, CHUNK // 16, grp, 0)

        def store_chunk(b, g):
            p0 = base + g * CHUNK
            jj = lax.shift_right_logical(p0, 12)
            ii = pl.multiple_of(lax.bitwise_and(p0, XR - 1), CHUNK)
            pltpu.async_copy(
                rows.at[b], out_hbm.at[jj, pl.ds(ii, CHUNK)], osem[b])

        fire_gathers(0, 0)

        def body(p, carry):
            @pl.when(p > 0)
            def _():
                wait_sem(osem[1])

            fire_gathers(1, 2 * p + 1)

            wait_sem(gsem[0])
            scale_chunk(0)
            store_chunk(0, 2 * p)

            wait_sem(gsem[1])
            scale_chunk(1)
            store_chunk(1, 2 * p + 1)

            @pl.when(p < NG // 2 - 1)
            def _():
                wait_sem(osem[0])
                fire_gathers(0, 2 * p + 2)

            return carry

        lax.fori_loop(0, NG // 2, body, 0)
        wait_sem(osem[0])
        wait_sem(osem[1])

    return k(xT, table)


def kernel(x, table):
    outT = _sc_embed(jnp.transpose(x), table)
    return jnp.transpose(outT, (1, 0, 2))


# speedup vs baseline: 1.0937x; 1.0610x over previous
import functools

import jax
import jax.numpy as jnp
from jax import lax
from jax.experimental import pallas as pl
from jax.experimental.pallas import tpu as pltpu
from jax.experimental.pallas import tpu_sc as plsc

D = 64
XR = 4096
COLS = 200
B = XR * COLS
NC = 2
NS = 16
NW = NC * NS
Q = B // 128
QPW = Q // NW
SUB = 128
CHUNK = 512
NSUB = CHUNK // SUB
NG = QPW // NSUB
SCALE = 8.0


def _fmt_idx(xT):
    mesh = plsc.VectorSubcoreMesh(
        core_axis_name="c", subcore_axis_name="s", num_cores=NC)

    @functools.partial(
        pl.kernel,
        mesh=mesh,
        compiler_params=pltpu.CompilerParams(use_tc_tiling_on_sc=True),
        out_type=jax.ShapeDtypeStruct((Q, SUB), jnp.int32),
        scratch_types=[
            pltpu.VMEM((8, SUB), jnp.int32),
            pltpu.SemaphoreType.DMA,
        ],
    )
    def ka(xT_hbm, out_hbm, stg, sem):
        wid = lax.axis_index("s") * NC + lax.axis_index("c")

        def tile_body(a, carry):
            pltpu.sync_copy(
                xT_hbm.at[pl.ds(pl.multiple_of(a * 8, 8), 8),
                          pl.ds(wid * SUB, SUB)], stg)
            cps = [
                pltpu.async_copy(
                    stg.at[r], out_hbm.at[(a * 8 + r) * NW + wid], sem)
                for r in range(8)
            ]
            for cp in cps:
                cp.wait()
            return carry

        lax.fori_loop(0, COLS // 8, tile_body, 0)

    return ka(xT)


def _sc_embed(xTr, table):
    mesh = plsc.VectorSubcoreMesh(
        core_axis_name="c", subcore_axis_name="s", num_cores=NC)

    @functools.partial(
        pl.kernel,
        mesh=mesh,
        compiler_params=pltpu.CompilerParams(use_tc_tiling_on_sc=False),
        out_type=jax.ShapeDtypeStruct((COLS, XR, D), jnp.float32),
        scratch_types=[
            pltpu.VMEM((2, NSUB, SUB), jnp.int32),
            pltpu.VMEM((2, CHUNK, D), jnp.float32),
            pltpu.SemaphoreType.DMA,
            pltpu.SemaphoreType.DMA,
            pltpu.SemaphoreType.DMA,
            pltpu.SemaphoreType.DMA,
        ],
    )
    def k(xTr_hbm, tbl_hbm, out_hbm, idx_v, rows, g0, g1, o0, o1):
        wid = lax.axis_index("s") * NC + lax.axis_index("c")
        qbase = wid * QPW
        gsem = (g0, g1)
        osem = (o0, o1)

        def fire_gathers(b, g):
            q0 = qbase + g * NSUB
            pltpu.sync_copy(xTr_hbm.at[pl.ds(q0, NSUB)], idx_v.at[b])
            for j in range(NSUB):
                pltpu.async_copy(
                    tbl_hbm.at[idx_v.at[b, j]],
                    rows.at[b, pl.ds(j * SUB, SUB), :], gsem[b])

        def wait_sem(sem):
            pltpu.make_async_copy(
                out_hbm.at[0, pl.ds(0, CHUNK)], rows.at[0], sem).wait()

        def scale_chunk(b):
            for j in range(NSUB):
                def grp(kk, cc, j=j):
                    iv = idx_v[b, j, pl.ds(kk * 16, 16)]
                    fv = jnp.where(iv == 0, jnp.float32(0.0),
                                   jnp.float32(SCALE))
                    for t in range(16):
                        f = fv[t]
                        rr = j * SUB + kk * 16 + t
                        for q in range(D // 16):
                            sl = pl.ds(q * 16, 16)
                            rows[b, rr, sl] = rows[b, rr, sl] * f
                    return cc

                lax.fori_loop(0, SUB // 16, grp, 0)

        def store_chunk(b, g):
            q0 = qbase + g * NSUB
            jj = q0 // NW
            ii = pl.multiple_of((q0 % NW) * SUB, CHUNK)
            pltpu.async_copy(
                rows.at[b], out_hbm.at[jj, pl.ds(ii, CHUNK)], osem[b])

        fire_gathers(0, 0)

        def body(p, carry):
            @pl.when(p > 0)
            def _():
                wait_sem(osem[1])

            fire_gathers(1, 2 * p + 1)

            wait_sem(gsem[0])
            scale_chunk(0)
            store_chunk(0, 2 * p)

            wait_sem(gsem[1])
            scale_chunk(1)
            store_chunk(1, 2 * p + 1)

            @pl.when(p < NG // 2 - 1)
            def _():
                wait_sem(osem[0])
                fire_gathers(0, 2 * p + 2)

            return carry

        lax.fori_loop(0, NG // 2, body, 0)
        wait_sem(osem[0])
        wait_sem(osem[1])

    return k(xTr, table)


def kernel(x, table):
    xTr = _fmt_idx(jnp.transpose(x))
    outT = _sc_embed(xTr, table)
    return jnp.transpose(outT, (1, 0, 2))
